# baseline (speedup 1.0000x reference)
import functools

import jax
import jax.numpy as jnp
from jax import lax
from jax.experimental import pallas as pl
from jax.experimental.pallas import tpu as pltpu
from jax.experimental.pallas import tpu_sc as plsc

_NUM_CORES = 2
_NUM_SUBCORES = 16
_NUM_WORKERS = _NUM_CORES * _NUM_SUBCORES
_LANES = 16


def _sc_gather_sum(idx_pad, emb_table, n_valid, rpw):
    emb = emb_table.shape[1]
    mesh = plsc.VectorSubcoreMesh(core_axis_name="c", subcore_axis_name="s")

    @functools.partial(
        pl.kernel,
        mesh=mesh,
        out_type=jax.ShapeDtypeStruct((_NUM_WORKERS, emb), jnp.float32),
        compiler_params=pltpu.CompilerParams(use_tc_tiling_on_sc=False),
        scratch_types=[
            pltpu.VMEM((rpw,), jnp.int32),
            pltpu.VMEM((rpw, emb), jnp.float32),
            pltpu.VMEM((emb,), jnp.float32),
            pltpu.SemaphoreType.DMA,
        ],
    )
    def gather_sum(idx_hbm, table_hbm, out_hbm, idx_v, rows_v, acc_v, sem):
        wid = lax.axis_index("s") * _NUM_CORES + lax.axis_index("c")
        base = wid * rpw
        pltpu.sync_copy(idx_hbm.at[pl.ds(base, rpw)], idx_v)
        pltpu.async_copy(table_hbm.at[idx_v], rows_v, sem).wait()
        for c in range(emb // _LANES):
            sl = pl.ds(c * _LANES, _LANES)
            s = rows_v[0, sl] * jnp.where(base + 0 < n_valid, 1.0, 0.0)
            for j in range(1, rpw):
                s = s + rows_v[j, sl] * jnp.where(base + j < n_valid, 1.0, 0.0)
            acc_v[sl] = s
        pltpu.sync_copy(acc_v, out_hbm.at[wid])

    return gather_sum(idx_pad, emb_table)


def _tc_body(img_ref, wimg_ref, part_ref, wemb_ref, b_ref, out_ref):
    s0 = jnp.sum(part_ref[...] * wemb_ref[...]) + b_ref[0, 0]
    scores = lax.dot_general(
        img_ref[...], wimg_ref[...], (((1,), (0,)), ((), ())),
        preferred_element_type=jnp.float32)
    scores = scores + s0
    m = jnp.max(scores)
    e = jnp.exp(scores - m)
    out_ref[...] = e * (1.0 / jnp.sum(e))


def kernel(text_input, image_input, emb_table, W, b):
    out_dim, img_dim = image_input.shape
    emb = emb_table.shape[1]

    idx = text_input.reshape(-1).astype(jnp.int32)
    n = idx.shape[0]
    rpw = -(-n // _NUM_WORKERS)
    rpw = -(-rpw // 8) * 8
    npad = _NUM_WORKERS * rpw
    if npad > n:
        idx = jnp.concatenate([idx, jnp.zeros((npad - n,), jnp.int32)])

    partials = _sc_gather_sum(idx, emb_table, n, rpw)

    w_emb = W[:emb].reshape(1, emb)
    w_img = W[emb:].reshape(img_dim, 1)
    b2 = b.reshape(1, 1)

    probs = pl.pallas_call(
        _tc_body,
        out_shape=jax.ShapeDtypeStruct((out_dim, 1), jnp.float32),
    )(image_input, w_img, partials, w_emb, b2)
    return probs.reshape(1, out_dim)


# speedup vs baseline: 1.7126x; 1.7126x over previous
import functools

import jax
import jax.numpy as jnp
from jax import lax
from jax.experimental import pallas as pl
from jax.experimental.pallas import tpu as pltpu
from jax.experimental.pallas import tpu_sc as plsc

_NUM_CORES = 2
_NUM_SUBCORES = 16
_NUM_WORKERS = _NUM_CORES * _NUM_SUBCORES
_LANES = 16


def _sc_gather_sum(idx_pad, emb_table, n_valid, rpw):
    emb = emb_table.shape[1]
    mesh = plsc.VectorSubcoreMesh(core_axis_name="c", subcore_axis_name="s")

    @functools.partial(
        pl.kernel,
        mesh=mesh,
        out_type=jax.ShapeDtypeStruct((_NUM_WORKERS, emb), jnp.float32),
        compiler_params=pltpu.CompilerParams(
            use_tc_tiling_on_sc=True, needs_layout_passes=False),
        scratch_types=[
            pltpu.VMEM((_LANES,), jnp.int32),
            pltpu.VMEM((rpw, 8, emb), jnp.float32),
            pltpu.VMEM((emb,), jnp.float32),
            pltpu.SemaphoreType.DMA,
        ],
    )
    def gather_sum(idx_hbm, table_hbm, out_hbm, idx_v, tiles_v, acc_v, sem):
        wid = lax.axis_index("s") * _NUM_CORES + lax.axis_index("c")
        base = wid * rpw
        pltpu.sync_copy(idx_hbm.at[pl.ds(base, _LANES)], idx_v)
        v16 = idx_v[...]
        lane = lax.iota(jnp.int32, _LANES)
        starts, rems = [], []
        for j in range(rpw):
            idx_j = jnp.sum(jnp.where(lane == j, v16, 0))
            rems.append(idx_j & 7)
            starts.append(pl.multiple_of(idx_j & ~7, 8))
        copies = [
            pltpu.async_copy(table_hbm.at[pl.ds(starts[j], 8)],
                             tiles_v.at[j], sem)
            for j in range(rpw)
        ]
        for c in copies:
            c.wait()
        accs = [jnp.zeros((_LANES,), jnp.float32) for _ in range(emb // _LANES)]
        for j in range(rpw):
            valid = jnp.where(base + j < n_valid, 1.0, 0.0)
            jv = jnp.full((_LANES,), j, jnp.int32)
            rv = jnp.broadcast_to(rems[j], (_LANES,))
            for c in range(emb // _LANES):
                row = plsc.load_gather(tiles_v, [jv, rv, lane + c * _LANES])
                accs[c] = accs[c] + row * valid
        for c in range(emb // _LANES):
            acc_v[pl.ds(c * _LANES, _LANES)] = accs[c]
        pltpu.sync_copy(acc_v, out_hbm.at[wid])

    return gather_sum(idx_pad, emb_table)


def _tc_body(img_ref, wimg_ref, part_ref, wemb_ref, b_ref, out_ref):
    s0 = jnp.sum(part_ref[...] * wemb_ref[...]) + b_ref[0, 0]
    scores = lax.dot_general(
        img_ref[...], wimg_ref[...], (((1,), (0,)), ((), ())),
        preferred_element_type=jnp.float32)
    scores = scores + s0
    m = jnp.max(scores)
    e = jnp.exp(scores - m)
    out_ref[...] = e * (1.0 / jnp.sum(e))


def kernel(text_input, image_input, emb_table, W, b):
    out_dim, img_dim = image_input.shape
    emb = emb_table.shape[1]

    idx = text_input.reshape(-1).astype(jnp.int32)
    n = idx.shape[0]
    rpw = -(-n // _NUM_WORKERS)
    rpw = -(-rpw // 8) * 8
    npad = _NUM_WORKERS * rpw + _LANES
    if npad > n:
        idx = jnp.concatenate([idx, jnp.zeros((npad - n,), jnp.int32)])

    partials = _sc_gather_sum(idx, emb_table, n, rpw)

    w_emb = W[:emb].reshape(1, emb)
    w_img = W[emb:].reshape(img_dim, 1)
    b2 = b.reshape(1, 1)

    probs = pl.pallas_call(
        _tc_body,
        out_shape=jax.ShapeDtypeStruct((out_dim, 1), jnp.float32),
    )(image_input, w_img, partials, w_emb, b2)
    return probs.reshape(1, out_dim)


# speedup vs baseline: 16.8820x; 9.8574x over previous
import functools

import jax
import jax.numpy as jnp
from jax import lax
from jax.experimental import pallas as pl
from jax.experimental.pallas import tpu as pltpu
from jax.experimental.pallas import tpu_sc as plsc

_NUM_CORES = 2
_NUM_SUBCORES = 16
_NUM_WORKERS = _NUM_CORES * _NUM_SUBCORES
_LANES = 16


def _sc_gather_sum(idx_pad, table_t, slab_t, n_valid, rpw):
    emb, vocab = table_t.shape
    tail_start = vocab - 128
    mesh = plsc.VectorSubcoreMesh(core_axis_name="c", subcore_axis_name="s")

    @functools.partial(
        pl.kernel,
        mesh=mesh,
        out_type=jax.ShapeDtypeStruct((_NUM_WORKERS, emb), jnp.float32),
        compiler_params=pltpu.CompilerParams(
            use_tc_tiling_on_sc=True, needs_layout_passes=False),
        scratch_types=[
            pltpu.VMEM((_LANES,), jnp.int32),
            pltpu.VMEM((rpw, emb, 128), jnp.float32),
            pltpu.VMEM((emb, 128), jnp.float32),
            pltpu.VMEM((emb,), jnp.float32),
            pltpu.SemaphoreType.DMA,
            pltpu.SemaphoreType.DMA,
        ],
    )
    def gather_sum(idx_hbm, table_hbm, slab_hbm, out_hbm,
                   idx_v, blocks_v, slab_v, acc_v, sem, sem2):
        wid = lax.axis_index("s") * _NUM_CORES + lax.axis_index("c")
        base = wid * rpw
        slab_cp = pltpu.async_copy(slab_hbm, slab_v, sem2)
        pltpu.sync_copy(idx_hbm.at[pl.ds(base, _LANES)], idx_v)
        v16 = idx_v[...]
        lane = lax.iota(jnp.int32, _LANES)
        idx_s, is_tail, offs = [], [], []
        copies = []
        for j in range(rpw):
            idx_j = jnp.sum(jnp.where(lane == j, v16, 0))
            tail_j = idx_j >= tail_start
            start_j = jnp.where(tail_j, 0, (idx_j >> 7) << 7)
            off_j = jnp.where(tail_j, idx_j - tail_start, idx_j & 127)
            idx_s.append(idx_j)
            is_tail.append(tail_j)
            offs.append(off_j)
            copies.append(pltpu.async_copy(
                table_hbm.at[:, pl.ds(pl.multiple_of(start_j, 128), 128)],
                blocks_v.at[j], sem))
        for c in copies:
            c.wait()
        slab_cp.wait()
        accs = [jnp.zeros((_LANES,), jnp.float32) for _ in range(emb // _LANES)]
        for j in range(rpw):
            valid = jnp.where(base + j < n_valid, 1.0, 0.0)
            jv = jnp.full((_LANES,), j, jnp.int32)
            ov = jnp.broadcast_to(offs[j], (_LANES,))
            tv = jnp.broadcast_to(is_tail[j], (_LANES,))
            for c in range(emb // _LANES):
                dim = lane + c * _LANES
                v_main = plsc.load_gather(blocks_v, [jv, dim, ov])
                v_tail = plsc.load_gather(slab_v, [dim, ov])
                accs[c] = accs[c] + jnp.where(tv, v_tail, v_main) * valid
        for c in range(emb // _LANES):
            acc_v[pl.ds(c * _LANES, _LANES)] = accs[c]
        pltpu.sync_copy(acc_v, out_hbm.at[wid])

    return gather_sum(idx_pad, table_t, slab_t)


def _tc_body(img_ref, wimg_ref, part_ref, wemb_ref, b_ref, out_ref):
    s0 = jnp.sum(part_ref[...] * wemb_ref[...]) + b_ref[0, 0]
    scores = lax.dot_general(
        img_ref[...], wimg_ref[...], (((1,), (0,)), ((), ())),
        preferred_element_type=jnp.float32)
    scores = scores + s0
    m = jnp.max(scores)
    e = jnp.exp(scores - m)
    out_ref[...] = e * (1.0 / jnp.sum(e))


def kernel(text_input, image_input, emb_table, W, b):
    out_dim, img_dim = image_input.shape
    emb = emb_table.shape[1]

    idx = text_input.reshape(-1).astype(jnp.int32)
    n = idx.shape[0]
    rpw = -(-n // _NUM_WORKERS)
    rpw = -(-rpw // 8) * 8
    npad = _NUM_WORKERS * rpw + _LANES
    if npad > n:
        idx = jnp.concatenate([idx, jnp.zeros((npad - n,), jnp.int32)])

    table_t = emb_table.T
    slab_t = table_t[:, emb_table.shape[0] - 128:]
    partials = _sc_gather_sum(idx, table_t, slab_t, n, rpw)

    w_emb = W[:emb].reshape(1, emb)
    w_img = W[emb:].reshape(img_dim, 1)
    b2 = b.reshape(1, 1)

    probs = pl.pallas_call(
        _tc_body,
        out_shape=jax.ShapeDtypeStruct((out_dim, 1), jnp.float32),
    )(image_input, w_img, partials, w_emb, b2)
    return probs.reshape(1, out_dim)


# speedup vs baseline: 20.6820x; 1.2251x over previous
import functools

import jax
import jax.numpy as jnp
from jax import lax
from jax.experimental import pallas as pl
from jax.experimental.pallas import tpu as pltpu
from jax.experimental.pallas import tpu_sc as plsc

_NUM_CORES = 2
_NUM_SUBCORES = 16
_NUM_WORKERS = _NUM_CORES * _NUM_SUBCORES
_LANES = 16


def _sc_gather_sum(idx, table_t, slab_t, n_valid, rpw):
    emb, vocab = table_t.shape
    tail_start = vocab - 128
    mesh = plsc.VectorSubcoreMesh(core_axis_name="c", subcore_axis_name="s")

    @functools.partial(
        pl.kernel,
        mesh=mesh,
        out_type=jax.ShapeDtypeStruct((_NUM_WORKERS, emb), jnp.float32),
        compiler_params=pltpu.CompilerParams(
            use_tc_tiling_on_sc=True, needs_layout_passes=False),
        scratch_types=[
            pltpu.VMEM((_LANES,), jnp.int32),
            pltpu.VMEM((rpw, emb, 128), jnp.float32),
            pltpu.VMEM((emb, 128), jnp.float32),
            pltpu.VMEM((emb,), jnp.float32),
            pltpu.SemaphoreType.DMA,
            pltpu.SemaphoreType.DMA,
        ],
    )
    def gather_sum(idx_hbm, table_hbm, slab_hbm, out_hbm,
                   idx_v, blocks_v, slab_v, acc_v, sem, sem2):
        wid = lax.axis_index("s") * _NUM_CORES + lax.axis_index("c")
        base = wid * rpw
        slab_cp = pltpu.async_copy(slab_hbm, slab_v, sem2)
        nload = min(rpw, 8)
        pltpu.sync_copy(idx_hbm.at[pl.ds(base, nload)],
                        idx_v.at[pl.ds(0, nload)])
        v16 = jnp.clip(idx_v[...], 0, vocab - 1)
        lane = lax.iota(jnp.int32, _LANES)
        is_tail, offs = [], []
        copies = []
        for j in range(rpw):
            idx_j = jnp.sum(jnp.where(lane == j, v16, 0))
            tail_j = idx_j >= tail_start
            start_j = jnp.where(tail_j, 0, (idx_j >> 7) << 7)
            off_j = jnp.where(tail_j, idx_j - tail_start, idx_j & 127)
            is_tail.append(tail_j)
            offs.append(off_j)
            copies.append(pltpu.async_copy(
                table_hbm.at[:, pl.ds(pl.multiple_of(start_j, 128), 128)],
                blocks_v.at[j], sem))
        for c in copies:
            c.wait()
        slab_cp.wait()
        accs = [jnp.zeros((_LANES,), jnp.float32) for _ in range(emb // _LANES)]
        for j in range(rpw):
            valid = jnp.where(base + j < n_valid, 1.0, 0.0)
            jv = jnp.full((_LANES,), j, jnp.int32)
            ov = jnp.broadcast_to(offs[j], (_LANES,))
            tv = jnp.broadcast_to(is_tail[j], (_LANES,))
            for c in range(emb // _LANES):
                dim = lane + c * _LANES
                v_main = plsc.load_gather(blocks_v, [jv, dim, ov])
                v_tail = plsc.load_gather(slab_v, [dim, ov])
                accs[c] = accs[c] + jnp.where(tv, v_tail, v_main) * valid
        for c in range(emb // _LANES):
            acc_v[pl.ds(c * _LANES, _LANES)] = accs[c]
        pltpu.sync_copy(acc_v, out_hbm.at[wid])

    return gather_sum(idx, table_t, slab_t)


def _tc_body(img_ref, wrow_ref, part_ref, b_ref, out_ref):
    emb = part_ref.shape[1]
    w_emb = wrow_ref[:, :emb]
    w_img = wrow_ref[:, emb:]
    s0 = jnp.sum(part_ref[...] * w_emb) + b_ref[0, 0]
    scores = lax.dot_general(
        w_img, img_ref[...], (((1,), (1,)), ((), ())),
        preferred_element_type=jnp.float32)
    scores = scores + s0
    m = jnp.max(scores)
    e = jnp.exp(scores - m)
    out_ref[...] = e * (1.0 / jnp.sum(e))


def kernel(text_input, image_input, emb_table, W, b):
    out_dim, img_dim = image_input.shape
    emb = emb_table.shape[1]

    idx = text_input.reshape(-1).astype(jnp.int32)
    n = idx.shape[0]
    rpw = -(-n // _NUM_WORKERS)
    rpw = -(-rpw // 8) * 8

    table_t = emb_table.T
    slab_t = table_t[:, emb_table.shape[0] - 128:]
    partials = _sc_gather_sum(idx, table_t, slab_t, n, rpw)

    w_row = W.T
    b2 = b.reshape(1, 1)

    return pl.pallas_call(
        _tc_body,
        out_shape=jax.ShapeDtypeStruct((1, out_dim), jnp.float32),
    )(image_input, w_row, partials, b2)


# speedup vs baseline: 23.6094x; 1.1415x over previous
import functools

import jax
import jax.numpy as jnp
from jax import lax
from jax.experimental import pallas as pl
from jax.experimental.pallas import tpu as pltpu
from jax.experimental.pallas import tpu_sc as plsc

_NUM_CORES = 2
_NUM_SUBCORES = 16
_NUM_WORKERS = _NUM_CORES * _NUM_SUBCORES
_LANES = 16


def _sc_gather_sum(idx, table_t, slab_t, n_valid, rpw):
    emb, vocab = table_t.shape
    tail_start = vocab - 128
    mesh = plsc.VectorSubcoreMesh(core_axis_name="c", subcore_axis_name="s")

    @functools.partial(
        pl.kernel,
        mesh=mesh,
        out_type=jax.ShapeDtypeStruct((_NUM_WORKERS, emb), jnp.float32),
        compiler_params=pltpu.CompilerParams(
            use_tc_tiling_on_sc=True, needs_layout_passes=False),
        scratch_types=[
            pltpu.VMEM((_LANES,), jnp.int32),
            pltpu.VMEM((rpw, emb, 128), jnp.float32),
            pltpu.VMEM((emb, 128), jnp.float32),
            pltpu.VMEM((emb,), jnp.float32),
            pltpu.SemaphoreType.DMA,
            pltpu.SemaphoreType.DMA,
        ],
    )
    def gather_sum(idx_hbm, table_hbm, slab_hbm, out_hbm,
                   idx_v, blocks_v, slab_v, acc_v, sem, sem2):
        wid = lax.axis_index("s") * _NUM_CORES + lax.axis_index("c")
        base = wid * rpw
        nload = min(rpw, 8)
        pltpu.sync_copy(idx_hbm.at[pl.ds(base, nload)],
                        idx_v.at[pl.ds(0, nload)])
        v16 = jnp.clip(idx_v[...], 0, vocab - 1)
        lane = lax.iota(jnp.int32, _LANES)
        masked = jnp.where(lane < rpw, v16, 0)
        any_tail = jnp.max(masked) >= tail_start
        slab_cp = pltpu.make_async_copy(slab_hbm, slab_v, sem2)

        @pl.when(any_tail)
        def _():
            slab_cp.start()

        is_tail, offs = [], []
        copies, valids = [], []
        for j in range(rpw):
            idx_j = jnp.sum(jnp.where(lane == j, v16, 0))
            tail_j = idx_j >= tail_start
            start_j = jnp.where(tail_j, 0, (idx_j >> 7) << 7)
            off_j = jnp.where(tail_j, idx_j - tail_start, idx_j & 127)
            is_tail.append(tail_j)
            offs.append(off_j)
            valid_j = base + j < n_valid
            valids.append(valid_j)
            cp = pltpu.make_async_copy(
                table_hbm.at[:, pl.ds(pl.multiple_of(start_j, 128), 128)],
                blocks_v.at[j], sem)
            copies.append(cp)

            @pl.when(valid_j)
            def _(cp=cp):
                cp.start()

        for j in range(rpw):

            @pl.when(valids[j])
            def _(cp=copies[j]):
                cp.wait()

        @pl.when(any_tail)
        def _():
            slab_cp.wait()
        accs = [jnp.zeros((_LANES,), jnp.float32) for _ in range(emb // _LANES)]
        for j in range(rpw):
            vv = jnp.broadcast_to(valids[j], (_LANES,))
            jv = jnp.full((_LANES,), j, jnp.int32)
            ov = jnp.broadcast_to(offs[j], (_LANES,))
            tv = jnp.broadcast_to(is_tail[j], (_LANES,))
            for c in range(emb // _LANES):
                dim = lane + c * _LANES
                v_main = plsc.load_gather(blocks_v, [jv, dim, ov])
                v_tail = plsc.load_gather(slab_v, [dim, ov])
                picked = jnp.where(tv, v_tail, v_main)
                accs[c] = accs[c] + jnp.where(vv, picked, 0.0)
        for c in range(emb // _LANES):
            acc_v[pl.ds(c * _LANES, _LANES)] = accs[c]
        pltpu.sync_copy(acc_v, out_hbm.at[wid])

    return gather_sum(idx, table_t, slab_t)


def _tc_matvec(img_ref, wrow_ref, b_ref, out_ref):
    emb = wrow_ref.shape[1] - img_ref.shape[1]
    w_img = wrow_ref[:, emb:]
    scores = lax.dot_general(
        w_img, img_ref[...], (((1,), (1,)), ((), ())),
        preferred_element_type=jnp.float32)
    out_ref[...] = scores + b_ref[0, 0]


def _tc_combine(scores_ref, part_ref, wrow_ref, out_ref):
    emb = part_ref.shape[1]
    s0 = jnp.sum(part_ref[...] * wrow_ref[:, :emb])
    scores = scores_ref[...] + s0
    m = jnp.max(scores)
    e = jnp.exp(scores - m)
    out_ref[...] = e * (1.0 / jnp.sum(e))


def kernel(text_input, image_input, emb_table, W, b):
    out_dim, img_dim = image_input.shape
    emb = emb_table.shape[1]

    idx = text_input.reshape(-1).astype(jnp.int32)
    n = idx.shape[0]
    rpw = -(-n // _NUM_WORKERS)
    rpw = -(-rpw // 8) * 8

    table_t = emb_table.T
    slab_t = table_t[:, emb_table.shape[0] - 128:]
    partials = _sc_gather_sum(idx, table_t, slab_t, n, rpw)

    w_row = W.T
    b2 = b.reshape(1, 1)

    scores = pl.pallas_call(
        _tc_matvec,
        out_shape=jax.ShapeDtypeStruct((1, out_dim), jnp.float32),
    )(image_input, w_row, b2)

    return pl.pallas_call(
        _tc_combine,
        out_shape=jax.ShapeDtypeStruct((1, out_dim), jnp.float32),
    )(scores, partials, w_row)
